# baseline (speedup 1.0000x reference)
import jax
import jax.numpy as jnp
from jax import lax
from jax.experimental import pallas as pl
from jax.experimental.pallas import tpu as pltpu

B, W, L = 16, 256, 20
J = L * L
BB = 4


def _cooc_kernel(nodes_ref, masks_ref, k_ref, out_ref,
                 s_ref, kbig_ref, veq_ref, kdrep_ref, eyew_ref):
    @pl.when(pl.program_id(0) == 0)
    def _build_constants():
        k = k_ref[...]
        prow = lax.broadcasted_iota(jnp.int32, (L, J), 0)
        jcol = lax.broadcasted_iota(jnp.int32, (L, J), 1)
        S = (prow == jcol // L).astype(jnp.float32)
        s_ref[...] = S.astype(jnp.bfloat16)

        kS = jnp.dot(k, S, preferred_element_type=jnp.float32)
        Krep = lax.dot_general(S, kS, (((0,), (0,)), ((), ())),
                               preferred_element_type=jnp.float32)
        vr = lax.broadcasted_iota(jnp.int32, (J, J), 0) % L
        vc = lax.broadcasted_iota(jnp.int32, (J, J), 1) % L
        veq = vr == vc
        kbig_ref[...] = jnp.where(veq, Krep, 0.0).astype(jnp.bfloat16)
        veq_ref[...] = veq.astype(jnp.bfloat16)

        eyeL = (lax.broadcasted_iota(jnp.int32, (L, L), 0) ==
                lax.broadcasted_iota(jnp.int32, (L, L), 1)).astype(jnp.float32)
        kdiag = jnp.sum(k * eyeL, axis=0, keepdims=True)
        kdrep_ref[...] = jnp.dot(kdiag, S,
                                 preferred_element_type=jnp.float32)
        eyew_ref[...] = (
            lax.broadcasted_iota(jnp.int32, (W, W), 0) ==
            lax.broadcasted_iota(jnp.int32, (W, W), 1)).astype(jnp.float32)

    S = s_ref[...]
    vidx = (lax.broadcasted_iota(jnp.int32, (W, J), 1) % L).astype(jnp.float32)
    eyeW = eyew_ref[...]
    ones_bf = jnp.ones((1, W), dtype=jnp.bfloat16)

    for i in range(BB):
        nodes = nodes_ref[i]
        masks = masks_ref[i]
        cval = (nodes + 32 * (1 - masks)).astype(jnp.bfloat16)
        crep = jnp.dot(cval, S, preferred_element_type=jnp.float32)
        X = (crep == vidx).astype(jnp.bfloat16)

        masksf = masks.astype(jnp.float32)
        lens_col = jnp.sum(masksf, axis=1, keepdims=True)
        rl_col = 1.0 / (lens_col + 1e-8)

        Y = jnp.dot(X, kbig_ref[...],
                    preferred_element_type=jnp.float32)
        Yn = (Y * rl_col).astype(jnp.bfloat16)
        Xf = X.astype(jnp.float32)
        Xn = (Xf * rl_col).astype(jnp.bfloat16)
        C = lax.dot_general(Yn, Xn, (((1,), (1,)), ((), ())),
                            preferred_element_type=jnp.float32)

        colsum = jnp.dot(ones_bf, X,
                         preferred_element_type=jnp.float32)
        counts_rep = jnp.dot(colsum.astype(jnp.bfloat16), veq_ref[...],
                             preferred_element_type=jnp.float32)
        svec = jnp.where(counts_rep == 1.0, kdrep_ref[...], 0.0)
        contrib = jnp.sum(Xf * svec, axis=1, keepdims=True)

        out_ref[i] = C - (contrib * rl_col * rl_col) * eyeW


def kernel(anonymized_nodes, walk_masks, kernel):
    k = kernel[:L, :L]
    return pl.pallas_call(
        _cooc_kernel,
        grid=(B // BB,),
        in_specs=[
            pl.BlockSpec((BB, W, L), lambda b: (b, 0, 0)),
            pl.BlockSpec((BB, W, L), lambda b: (b, 0, 0)),
            pl.BlockSpec((L, L), lambda b: (0, 0)),
        ],
        out_specs=pl.BlockSpec((BB, W, W), lambda b: (b, 0, 0)),
        out_shape=jax.ShapeDtypeStruct((B, W, W), jnp.float32),
        scratch_shapes=[
            pltpu.VMEM((L, J), jnp.bfloat16),
            pltpu.VMEM((J, J), jnp.bfloat16),
            pltpu.VMEM((J, J), jnp.bfloat16),
            pltpu.VMEM((1, J), jnp.float32),
            pltpu.VMEM((W, W), jnp.float32),
        ],
    )(anonymized_nodes, walk_masks, k)


# speedup vs baseline: 1.0463x; 1.0463x over previous
import jax
import jax.numpy as jnp
from jax import lax
from jax.experimental import pallas as pl
from jax.experimental.pallas import tpu as pltpu

B, W, L = 16, 256, 20
J = L * L
BB = 4


def _cooc_kernel(nodes_ref, masks_ref, k_ref, out_ref,
                 s_ref, kbig_ref, veq_ref, kdrep_ref):
    @pl.when(pl.program_id(0) == 0)
    def _build_constants():
        k = k_ref[...]
        prow = lax.broadcasted_iota(jnp.int32, (L, J), 0)
        jcol = lax.broadcasted_iota(jnp.int32, (L, J), 1)
        S = (prow == jcol // L).astype(jnp.float32)
        s_ref[...] = S.astype(jnp.bfloat16)

        kS = jnp.dot(k, S, preferred_element_type=jnp.float32)
        Krep = lax.dot_general(S, kS, (((0,), (0,)), ((), ())),
                               preferred_element_type=jnp.float32)
        vr = lax.broadcasted_iota(jnp.int32, (J, J), 0) % L
        vc = lax.broadcasted_iota(jnp.int32, (J, J), 1) % L
        veq = vr == vc
        kbig_ref[...] = jnp.where(veq, Krep, 0.0).astype(jnp.bfloat16)
        veq_ref[...] = veq.astype(jnp.float32)

        eyeL = (lax.broadcasted_iota(jnp.int32, (L, L), 0) ==
                lax.broadcasted_iota(jnp.int32, (L, L), 1)).astype(jnp.float32)
        kdiag = jnp.sum(k * eyeL, axis=0, keepdims=True)
        kdrep_ref[...] = jnp.dot(kdiag, S,
                                 preferred_element_type=jnp.float32)

    S = s_ref[...]
    vidx = (lax.broadcasted_iota(jnp.int32, (W, J), 1) % L).astype(jnp.float32)
    eyeW = (lax.broadcasted_iota(jnp.int32, (W, W), 0) ==
            lax.broadcasted_iota(jnp.int32, (W, W), 1)).astype(jnp.float32)

    for i in range(BB):
        nodes = nodes_ref[i].astype(jnp.bfloat16)
        masksf = masks_ref[i].astype(jnp.float32)

        nrep = jnp.dot(nodes, S, preferred_element_type=jnp.float32)
        mrep = jnp.dot(masksf.astype(jnp.bfloat16), S,
                       preferred_element_type=jnp.float32)
        X = jnp.where(nrep == vidx, mrep, 0.0).astype(jnp.bfloat16)

        lens_col = jnp.sum(masksf, axis=1, keepdims=True)
        rl_col = (1.0 / (lens_col + 1e-8)).astype(jnp.float32)

        Y = jnp.dot(X, kbig_ref[...],
                    preferred_element_type=jnp.float32)
        Yn = (Y * rl_col).astype(jnp.bfloat16)
        Xn = (X.astype(jnp.float32) * rl_col).astype(jnp.bfloat16)
        C = lax.dot_general(Yn, Xn, (((1,), (1,)), ((), ())),
                            preferred_element_type=jnp.float32)

        X32 = X.astype(jnp.float32)
        colsum = jnp.sum(X32, axis=0, keepdims=True)
        counts_rep = jnp.dot(colsum, veq_ref[...],
                             preferred_element_type=jnp.float32)
        svec = jnp.where(counts_rep == 1.0, kdrep_ref[...], 0.0)
        contrib = jnp.sum(X32 * svec, axis=1, keepdims=True)

        out_ref[i] = C - (contrib * rl_col * rl_col) * eyeW


def kernel(anonymized_nodes, walk_masks, kernel):
    return pl.pallas_call(
        _cooc_kernel,
        grid=(B // BB,),
        in_specs=[
            pl.BlockSpec((BB, W, L), lambda b: (b, 0, 0)),
            pl.BlockSpec((BB, W, L), lambda b: (b, 0, 0)),
            pl.BlockSpec((L, L), lambda b: (0, 0)),
        ],
        out_specs=pl.BlockSpec((BB, W, W), lambda b: (b, 0, 0)),
        out_shape=jax.ShapeDtypeStruct((B, W, W), jnp.float32),
        scratch_shapes=[
            pltpu.VMEM((L, J), jnp.bfloat16),
            pltpu.VMEM((J, J), jnp.bfloat16),
            pltpu.VMEM((J, J), jnp.float32),
            pltpu.VMEM((1, J), jnp.float32),
        ],
    )(anonymized_nodes, walk_masks, kernel)


# speedup vs baseline: 1.1181x; 1.0686x over previous
import jax
import jax.numpy as jnp
from jax import lax
from jax.experimental import pallas as pl
from jax.experimental.pallas import tpu as pltpu

B, W, L = 16, 256, 20
J = L * L
BB = 4


def _cooc_kernel(nodes_ref, masks_ref, k_ref, out_ref,
                 s_ref, kbig_ref, veq_ref, kdrep_ref):
    @pl.when(pl.program_id(0) == 0)
    def _build_constants():
        k = k_ref[...]
        prow = lax.broadcasted_iota(jnp.int32, (L, J), 0)
        jcol = lax.broadcasted_iota(jnp.int32, (L, J), 1)
        S = (prow == jcol // L).astype(jnp.float32)
        s_ref[...] = S.astype(jnp.bfloat16)

        kS = jnp.dot(k, S, preferred_element_type=jnp.float32)
        Krep = lax.dot_general(S, kS, (((0,), (0,)), ((), ())),
                               preferred_element_type=jnp.float32)
        vr = lax.broadcasted_iota(jnp.int32, (J, J), 0) % L
        vc = lax.broadcasted_iota(jnp.int32, (J, J), 1) % L
        veq = vr == vc
        kbig_ref[...] = jnp.where(veq, Krep, 0.0).astype(jnp.bfloat16)
        veq_ref[...] = veq.astype(jnp.float32)

        eyeL = (lax.broadcasted_iota(jnp.int32, (L, L), 0) ==
                lax.broadcasted_iota(jnp.int32, (L, L), 1)).astype(jnp.float32)
        kdiag = jnp.sum(k * eyeL, axis=0, keepdims=True)
        kdrep_ref[...] = jnp.dot(kdiag, S,
                                 preferred_element_type=jnp.float32)

    S = s_ref[...]
    vidx = (lax.broadcasted_iota(jnp.int32, (W, J), 1) % L).astype(jnp.float32)
    eyeW = (lax.broadcasted_iota(jnp.int32, (W, W), 0) ==
            lax.broadcasted_iota(jnp.int32, (W, W), 1)).astype(jnp.float32)

    for i in range(BB):
        nodes = nodes_ref[i]
        masks = masks_ref[i]
        cval = (nodes + 32 * (1 - masks)).astype(jnp.bfloat16)
        crep = jnp.dot(cval, S, preferred_element_type=jnp.float32)

        masksf = masks.astype(jnp.float32)
        lens_col = jnp.sum(masksf, axis=1, keepdims=True)
        rl_col = 1.0 / (lens_col + 1e-8)

        X32 = jnp.where(crep == vidx, 1.0, 0.0)
        Xr = (X32 * rl_col).astype(jnp.bfloat16)

        Y = jnp.dot(Xr, kbig_ref[...],
                    preferred_element_type=jnp.float32)
        Yn = Y.astype(jnp.bfloat16)
        C = lax.dot_general(Yn, Xr, (((1,), (1,)), ((), ())),
                            preferred_element_type=jnp.float32)

        colsum = jnp.sum(X32, axis=0, keepdims=True)
        counts_rep = jnp.dot(colsum, veq_ref[...],
                             preferred_element_type=jnp.float32)
        svec = jnp.where(counts_rep == 1.0, kdrep_ref[...], 0.0)
        contrib = jnp.sum(X32 * svec, axis=1, keepdims=True)

        out_ref[i] = C - (contrib * rl_col * rl_col) * eyeW


def kernel(anonymized_nodes, walk_masks, kernel):
    return pl.pallas_call(
        _cooc_kernel,
        grid=(B // BB,),
        in_specs=[
            pl.BlockSpec((BB, W, L), lambda b: (b, 0, 0)),
            pl.BlockSpec((BB, W, L), lambda b: (b, 0, 0)),
            pl.BlockSpec((L, L), lambda b: (0, 0)),
        ],
        out_specs=pl.BlockSpec((BB, W, W), lambda b: (b, 0, 0)),
        out_shape=jax.ShapeDtypeStruct((B, W, W), jnp.float32),
        scratch_shapes=[
            pltpu.VMEM((L, J), jnp.bfloat16),
            pltpu.VMEM((J, J), jnp.bfloat16),
            pltpu.VMEM((J, J), jnp.float32),
            pltpu.VMEM((1, J), jnp.float32),
        ],
    )(anonymized_nodes, walk_masks, kernel)
